# baseline (speedup 1.0000x reference)
import functools

import jax
import jax.numpy as jnp
from jax import lax
from jax.experimental import pallas as pl
from jax.experimental.pallas import tpu as pltpu
from jax.experimental.pallas import tpu_sc as plsc

_C = 64
_NX = 512
_NY = 512
_NXY = _NX * _NY
_P = 30000
_PPAD = 4096
_NC = 2
_NS = 16
_NW = _NC * _NS
_COLS_PER_W = _NXY // _NW
_CHUNK = 256
_NCHUNK = _COLS_PER_W // _CHUNK
_L = 16
_SLOT = 1920
_PTOT = _SLOT * _NS


def _sc_scatter(coords, pfpad):
    mesh = plsc.VectorSubcoreMesh(core_axis_name="c", subcore_axis_name="s")

    @functools.partial(
        pl.kernel,
        mesh=mesh,
        compiler_params=pltpu.CompilerParams(
            needs_layout_passes=False, use_tc_tiling_on_sc=False),
        out_type=(
            jax.ShapeDtypeStruct((_C, _NXY), jnp.float32),
            jax.ShapeDtypeStruct((_NC, 2, _PTOT), jnp.int32),
        ),
        scratch_types=[
            pltpu.VMEM((_COLS_PER_W,), jnp.int32),
            pltpu.VMEM((_SLOT * 4,), jnp.int32),
            pltpu.VMEM((_SLOT,), jnp.int32),
            pltpu.VMEM((_SLOT,), jnp.int32),
            pltpu.VMEM((_SLOT,), jnp.int32),
            pltpu.VMEM((_SLOT,), jnp.int32),
            pltpu.VMEM((_CHUNK,), jnp.int32),
            pltpu.VMEM((_CHUNK,), jnp.int32),
            pltpu.VMEM((_CHUNK, _C), jnp.float32),
            pltpu.VMEM((_CHUNK, _C), jnp.float32),
            pltpu.VMEM((_C, _CHUNK), jnp.float32),
            pltpu.VMEM((_C, _CHUNK), jnp.float32),
            pltpu.SemaphoreType.DMA,
            pltpu.SemaphoreType.DMA,
            pltpu.SemaphoreType.DMA,
            pltpu.SemaphoreType.DMA,
            pltpu.SemaphoreType.DMA,
            pltpu.SemaphoreType.DMA,
        ],
    )
    def k(coords_hbm, pf_hbm, out_hbm, rt_hbm, win_v, crd_v,
          own0, own1, key0, key1, pl0, pl1, rows0, rows1, cm0, cm1,
          s0, s1, s2, s3, s4, s5):
        scid = lax.axis_index("c")
        sid = lax.axis_index("s")
        wid = sid * _NC + scid
        lo = wid * _COLS_PER_W
        lane = jnp.arange(_L, dtype=jnp.int32)
        ownb = (own0, own1)
        keyb = (key0, key1)
        plist = (pl0, pl1)
        rows = (rows0, rows1)
        cmb = (cm0, cm1)

        def init_body(i, _):
            win_v[pl.ds(i * _L, _L)] = jnp.full((_L,), -1, jnp.int32)
            return 0

        lax.fori_loop(0, _COLS_PER_W // _L, init_body, 0)

        g0 = sid * _SLOT
        n = jnp.minimum(_SLOT, _P - g0)
        pltpu.async_copy(
            coords_hbm.at[pl.ds(g0 * 4, _SLOT * 4)], crd_v, s0).wait()

        def prep_body(vi, _):
            r16 = vi * _L + lane
            c2 = plsc.load_gather(crd_v, [r16 * 4 + 2])
            c3 = plsc.load_gather(crd_v, [r16 * 4 + 3])
            idx = c2 * _NX + c3
            valid = r16 < n
            owner = jnp.where(valid, idx >> 13, -1)
            key = ((idx & (_COLS_PER_W - 1)) << 15) | (g0 + r16)
            own0[pl.ds(vi * _L, _L)] = owner
            key0[pl.ds(vi * _L, _L)] = key
            return 0

        lax.fori_loop(0, _SLOT // _L, prep_body, 0)
        pltpu.sync_copy(own0, rt_hbm.at[scid, 0].at[pl.ds(g0, _SLOT)])
        pltpu.sync_copy(key0, rt_hbm.at[scid, 1].at[pl.ds(g0, _SLOT)])
        plsc.subcore_barrier()

        def stage_rt(ci):
            b = ci % 2
            ha = pltpu.async_copy(
                rt_hbm.at[scid, 0].at[pl.ds(ci * _SLOT, _SLOT)], ownb[b], s0)
            hb = pltpu.async_copy(
                rt_hbm.at[scid, 1].at[pl.ds(ci * _SLOT, _SLOT)], keyb[b], s1)
            return (ha, hb)

        def scan_chunk(ci):
            b = ci % 2

            def vec_body(vi, _):
                ow = ownb[b][pl.ds(vi * _L, _L)]
                key = keyb[b][pl.ds(vi * _L, _L)]
                m = ow == wid
                idxl = key >> 15

                def rmw(_go):
                    cur = plsc.load_gather(win_v, [idxl], mask=m)
                    plsc.store_scatter(win_v, [idxl], jnp.maximum(cur, key),
                                       mask=m)
                    chk = plsc.load_gather(win_v, [idxl], mask=m)
                    return jnp.any(m & (chk < key))

                lax.while_loop(lambda g: g, rmw, jnp.bool_(True))
                return 0

            lax.fori_loop(0, _SLOT // _L, vec_body, 0)

        h = stage_rt(0)
        for ci in range(_NS):
            h[0].wait()
            h[1].wait()
            if ci + 1 < _NS:
                h = stage_rt(ci + 1)
            scan_chunk(ci)

        def build(kk, b):
            cbase = kk * _CHUNK

            def bbody(vi, _):
                w = win_v[pl.ds(cbase + vi * _L, _L)]
                pwin = w & 32767
                col16 = cbase + vi * _L + lane
                dummy = _P + (col16 & (_PPAD - 1))
                plist[b][pl.ds(vi * _L, _L)] = jnp.where(w >= 0, pwin, dummy)
                return 0

            lax.fori_loop(0, _CHUNK // _L, bbody, 0)

        def gather_copy(b):
            return pltpu.make_async_copy(pf_hbm.at[plist[b]], rows[b],
                                         (s2, s3)[b])

        def write_copy(kk, b):
            return pltpu.make_async_copy(
                cmb[b], out_hbm.at[:, pl.ds(lo + kk * _CHUNK, _CHUNK)],
                (s4, s5)[b])

        def xpose(b):
            def xbody(jv, _):
                jvec = jv * _L + lane
                for c in range(_C):
                    v = plsc.load_gather(
                        rows[b], [jvec, jnp.full((_L,), c, jnp.int32)])
                    cmb[b][c, pl.ds(jv * _L, _L)] = v
                return 0

            lax.fori_loop(0, _CHUNK // _L, xbody, 0)

        def step(kk, b):
            gather_copy(b).wait()

            @pl.when(kk >= 1)
            def _():
                write_copy(kk - 1, 1 - b).wait()

            @pl.when(kk + 1 < _NCHUNK)
            def _():
                build(kk + 1, 1 - b)
                gather_copy(1 - b).start()

            xpose(b)
            write_copy(kk, b).start()

        build(0, 0)
        gather_copy(0).start()

        def pair_body(jj, _):
            step(2 * jj, 0)
            step(2 * jj + 1, 1)
            return 0

        lax.fori_loop(0, _NCHUNK // 2, pair_body, 0)
        write_copy(_NCHUNK - 1, 1).wait()

    return k(coords, pfpad)


def kernel(pillar_features, coords):
    pfpad = jnp.concatenate(
        [pillar_features, jnp.zeros((_PPAD, _C), jnp.float32)], axis=0)
    cflat = coords.astype(jnp.int32).reshape(-1)
    cpad = jnp.concatenate(
        [cflat, jnp.zeros(((_PTOT - _P) * 4,), jnp.int32)])
    out, _ = _sc_scatter(cpad, pfpad)
    return out.reshape(1, _C, _NY, _NX)


# speedup vs baseline: 1.7102x; 1.7102x over previous
import functools

import jax
import jax.numpy as jnp
from jax import lax
from jax.experimental import pallas as pl
from jax.experimental.pallas import tpu as pltpu
from jax.experimental.pallas import tpu_sc as plsc

_C = 64
_NX = 512
_NY = 512
_NXY = _NX * _NY
_P = 30000
_PPAD = 32768
_NC = 2
_NS = 16
_NW = _NC * _NS
_COLS_PER_W = _NXY // _NW
_CHUNK = 512
_NCHUNK = _COLS_PER_W // _CHUNK
_L = 16
_SLOT = 1920
_PTOT = _SLOT * _NS


def _sc_scatter(coords, pfpad):
    mesh = plsc.VectorSubcoreMesh(core_axis_name="c", subcore_axis_name="s")

    @functools.partial(
        pl.kernel,
        mesh=mesh,
        compiler_params=pltpu.CompilerParams(
            needs_layout_passes=False, use_tc_tiling_on_sc=False),
        out_type=(
            jax.ShapeDtypeStruct((_NXY, _C), jnp.float32),
            jax.ShapeDtypeStruct((_NC, 2, _PTOT), jnp.int32),
        ),
        scratch_types=[
            pltpu.VMEM((_COLS_PER_W,), jnp.int32),
            pltpu.VMEM((_SLOT * 4,), jnp.int32),
            pltpu.VMEM((_SLOT,), jnp.int32),
            pltpu.VMEM((_SLOT,), jnp.int32),
            pltpu.VMEM((_SLOT,), jnp.int32),
            pltpu.VMEM((_SLOT,), jnp.int32),
            pltpu.VMEM((_CHUNK,), jnp.int32),
            pltpu.VMEM((_CHUNK,), jnp.int32),
            pltpu.VMEM((_CHUNK, _C), jnp.float32),
            pltpu.VMEM((_CHUNK, _C), jnp.float32),
            pltpu.SemaphoreType.DMA,
            pltpu.SemaphoreType.DMA,
            pltpu.SemaphoreType.DMA,
            pltpu.SemaphoreType.DMA,
            pltpu.SemaphoreType.DMA,
            pltpu.SemaphoreType.DMA,
        ],
    )
    def k(coords_hbm, pf_hbm, out_hbm, rt_hbm, win_v, crd_v,
          own0, own1, key0, key1, pl0, pl1, rows0, rows1,
          s0, s1, s2, s3, s4, s5):
        scid = lax.axis_index("c")
        sid = lax.axis_index("s")
        wid = sid * _NC + scid
        lo = wid * _COLS_PER_W
        lane = jnp.arange(_L, dtype=jnp.int32)
        ownb = (own0, own1)
        keyb = (key0, key1)
        plist = (pl0, pl1)
        rows = (rows0, rows1)

        def init_body(i, _):
            win_v[pl.ds(i * _L, _L)] = jnp.full((_L,), -1, jnp.int32)
            return 0

        lax.fori_loop(0, _COLS_PER_W // _L, init_body, 0)

        g0 = sid * _SLOT
        n = jnp.minimum(_SLOT, _P - g0)
        pltpu.async_copy(
            coords_hbm.at[pl.ds(g0 * 4, _SLOT * 4)], crd_v, s0).wait()

        def prep_body(vi, _):
            r16 = vi * _L + lane
            c2 = plsc.load_gather(crd_v, [r16 * 4 + 2])
            c3 = plsc.load_gather(crd_v, [r16 * 4 + 3])
            idx = c2 * _NX + c3
            valid = r16 < n
            owner = jnp.where(valid, idx >> 13, -1)
            key = ((idx & (_COLS_PER_W - 1)) << 15) | (g0 + r16)
            own0[pl.ds(vi * _L, _L)] = owner
            key0[pl.ds(vi * _L, _L)] = key
            return 0

        lax.fori_loop(0, _SLOT // _L, prep_body, 0)
        pltpu.sync_copy(own0, rt_hbm.at[scid, 0].at[pl.ds(g0, _SLOT)])
        pltpu.sync_copy(key0, rt_hbm.at[scid, 1].at[pl.ds(g0, _SLOT)])
        plsc.subcore_barrier()

        def stage_rt(ci):
            b = ci % 2
            ha = pltpu.async_copy(
                rt_hbm.at[scid, 0].at[pl.ds(ci * _SLOT, _SLOT)], ownb[b], s0)
            hb = pltpu.async_copy(
                rt_hbm.at[scid, 1].at[pl.ds(ci * _SLOT, _SLOT)], keyb[b], s1)
            return (ha, hb)

        def scan_chunk(ci):
            b = ci % 2

            def vec_body(vi, _):
                ow = ownb[b][pl.ds(vi * _L, _L)]
                key = keyb[b][pl.ds(vi * _L, _L)]
                m = ow == wid
                idxl = key >> 15

                def rmw(_go):
                    cur = plsc.load_gather(win_v, [idxl], mask=m)
                    plsc.store_scatter(win_v, [idxl], jnp.maximum(cur, key),
                                       mask=m)
                    chk = plsc.load_gather(win_v, [idxl], mask=m)
                    return jnp.any(m & (chk < key))

                lax.while_loop(lambda g: g, rmw, jnp.bool_(True))
                return 0

            lax.fori_loop(0, _SLOT // _L, vec_body, 0)

        h = stage_rt(0)
        for ci in range(_NS):
            h[0].wait()
            h[1].wait()
            if ci + 1 < _NS:
                h = stage_rt(ci + 1)
            scan_chunk(ci)

        def build(kk):
            b = kk % 2
            cbase = kk * _CHUNK

            def bbody(vi, _):
                w = win_v[pl.ds(cbase + vi * _L, _L)]
                pwin = w & 32767
                col16 = lo + cbase + vi * _L + lane
                dummy = _P + (col16 & (_PPAD - 1))
                plist[b][pl.ds(vi * _L, _L)] = jnp.where(w >= 0, pwin, dummy)
                return 0

            lax.fori_loop(0, _CHUNK // _L, bbody, 0)

        def fire_gather(kk):
            b = kk % 2
            return pltpu.async_copy(pf_hbm.at[plist[b]], rows[b],
                                    (s2, s3)[b])

        def fire_write(kk):
            b = kk % 2
            return pltpu.async_copy(
                rows[b], out_hbm.at[pl.ds(lo + kk * _CHUNK, _CHUNK)],
                (s4, s5)[b])

        build(0)
        gh = {0: fire_gather(0)}
        wh = {}
        for kk in range(_NCHUNK):
            gh.pop(kk).wait()
            if kk >= 1:
                wh.pop(kk - 1).wait()
            if kk + 1 < _NCHUNK:
                build(kk + 1)
                gh[kk + 1] = fire_gather(kk + 1)
            wh[kk] = fire_write(kk)
        wh.pop(_NCHUNK - 1).wait()

    return k(coords, pfpad)


def kernel(pillar_features, coords):
    pfpad = jnp.concatenate(
        [pillar_features, jnp.zeros((_PPAD, _C), jnp.float32)], axis=0)
    cflat = coords.astype(jnp.int32).reshape(-1)
    cpad = jnp.concatenate(
        [cflat, jnp.zeros(((_PTOT - _P) * 4,), jnp.int32)])
    out_t, _ = _sc_scatter(cpad, pfpad)
    return out_t.T.reshape(1, _C, _NY, _NX)


# speedup vs baseline: 1.7120x; 1.0010x over previous
import functools

import jax
import jax.numpy as jnp
from jax import lax
from jax.experimental import pallas as pl
from jax.experimental.pallas import tpu as pltpu
from jax.experimental.pallas import tpu_sc as plsc

_C = 64
_NX = 512
_NY = 512
_NXY = _NX * _NY
_P = 30000
_PPAD = 32768
_NC = 2
_NS = 16
_NW = _NC * _NS
_COLS_PER_W = _NXY // _NW
_CHUNK = 512
_NCHUNK = _COLS_PER_W // _CHUNK
_L = 16
_SLOT = 1920
_PTOT = _SLOT * _NS


def _sc_scatter(coords, pfpad):
    mesh = plsc.VectorSubcoreMesh(core_axis_name="c", subcore_axis_name="s")

    @functools.partial(
        pl.kernel,
        mesh=mesh,
        compiler_params=pltpu.CompilerParams(
            needs_layout_passes=False, use_tc_tiling_on_sc=False),
        out_type=(
            jax.ShapeDtypeStruct((_NXY, _C), jnp.float32),
            jax.ShapeDtypeStruct((_NC, 2, _PTOT), jnp.int32),
        ),
        scratch_types=[
            pltpu.VMEM((_COLS_PER_W,), jnp.int32),
            pltpu.VMEM((_SLOT * 4,), jnp.int32),
            pltpu.VMEM((_SLOT,), jnp.int32),
            pltpu.VMEM((_SLOT,), jnp.int32),
            pltpu.VMEM((_SLOT,), jnp.int32),
            pltpu.VMEM((_SLOT,), jnp.int32),
            pltpu.VMEM((_CHUNK,), jnp.int32),
            pltpu.VMEM((_CHUNK,), jnp.int32),
            pltpu.VMEM((_CHUNK, _C), jnp.float32),
            pltpu.VMEM((_CHUNK, _C), jnp.float32),
            pltpu.SemaphoreType.DMA,
            pltpu.SemaphoreType.DMA,
            pltpu.SemaphoreType.DMA,
            pltpu.SemaphoreType.DMA,
            pltpu.SemaphoreType.DMA,
            pltpu.SemaphoreType.DMA,
        ],
    )
    def k(coords_hbm, pf_hbm, out_hbm, rt_hbm, win_v, crd_v,
          own0, own1, key0, key1, pl0, pl1, rows0, rows1,
          s0, s1, s2, s3, s4, s5):
        scid = lax.axis_index("c")
        sid = lax.axis_index("s")
        wid = sid * _NC + scid
        lo = wid * _COLS_PER_W
        lane = jnp.arange(_L, dtype=jnp.int32)
        ownb = (own0, own1)
        keyb = (key0, key1)
        plist = (pl0, pl1)
        rows = (rows0, rows1)

        def init_body(i, _):
            win_v[pl.ds(i * _L, _L)] = jnp.full((_L,), -1, jnp.int32)
            return 0

        lax.fori_loop(0, _COLS_PER_W // _L, init_body, 0)

        g0 = sid * _SLOT
        b0 = jnp.minimum(g0, _P - _SLOT)
        pltpu.async_copy(
            coords_hbm.at[pl.ds(b0 * 4, _SLOT * 4)], crd_v, s0).wait()

        def prep_body(vi, _):
            r16 = vi * _L + lane
            c2 = plsc.load_gather(crd_v, [r16 * 4 + 2])
            c3 = plsc.load_gather(crd_v, [r16 * 4 + 3])
            idx = c2 * _NX + c3
            gp = b0 + r16
            valid = gp >= g0
            owner = jnp.where(valid, idx >> 13, -1)
            key = ((idx & (_COLS_PER_W - 1)) << 15) | gp
            own0[pl.ds(vi * _L, _L)] = owner
            key0[pl.ds(vi * _L, _L)] = key
            return 0

        lax.fori_loop(0, _SLOT // _L, prep_body, 0)
        pltpu.sync_copy(own0, rt_hbm.at[scid, 0].at[pl.ds(g0, _SLOT)])
        pltpu.sync_copy(key0, rt_hbm.at[scid, 1].at[pl.ds(g0, _SLOT)])
        plsc.subcore_barrier()

        def stage_rt(ci):
            b = ci % 2
            ha = pltpu.async_copy(
                rt_hbm.at[scid, 0].at[pl.ds(ci * _SLOT, _SLOT)], ownb[b], s0)
            hb = pltpu.async_copy(
                rt_hbm.at[scid, 1].at[pl.ds(ci * _SLOT, _SLOT)], keyb[b], s1)
            return (ha, hb)

        def scan_chunk(ci):
            b = ci % 2

            def vec_body(vi, _):
                ow = ownb[b][pl.ds(vi * _L, _L)]
                key = keyb[b][pl.ds(vi * _L, _L)]
                m = ow == wid
                idxl = key >> 15

                def rmw(_go):
                    cur = plsc.load_gather(win_v, [idxl], mask=m)
                    plsc.store_scatter(win_v, [idxl], jnp.maximum(cur, key),
                                       mask=m)
                    chk = plsc.load_gather(win_v, [idxl], mask=m)
                    return jnp.any(m & (chk < key))

                lax.while_loop(lambda g: g, rmw, jnp.bool_(True))
                return 0

            lax.fori_loop(0, _SLOT // _L, vec_body, 0)

        h = stage_rt(0)
        for ci in range(_NS):
            h[0].wait()
            h[1].wait()
            if ci + 1 < _NS:
                h = stage_rt(ci + 1)
            scan_chunk(ci)

        def build(kk):
            b = kk % 2
            cbase = kk * _CHUNK

            def bbody(vi, _):
                w = win_v[pl.ds(cbase + vi * _L, _L)]
                pwin = w & 32767
                col16 = lo + cbase + vi * _L + lane
                dummy = _P + (col16 & (_PPAD - 1))
                plist[b][pl.ds(vi * _L, _L)] = jnp.where(w >= 0, pwin, dummy)
                return 0

            lax.fori_loop(0, _CHUNK // _L, bbody, 0)

        def fire_gather(kk):
            b = kk % 2
            return pltpu.async_copy(pf_hbm.at[plist[b]], rows[b],
                                    (s2, s3)[b])

        def fire_write(kk):
            b = kk % 2
            return pltpu.async_copy(
                rows[b], out_hbm.at[pl.ds(lo + kk * _CHUNK, _CHUNK)],
                (s4, s5)[b])

        build(0)
        gh = {0: fire_gather(0)}
        wh = {}
        for kk in range(_NCHUNK):
            gh.pop(kk).wait()
            if kk >= 1:
                wh.pop(kk - 1).wait()
            if kk + 1 < _NCHUNK:
                build(kk + 1)
                gh[kk + 1] = fire_gather(kk + 1)
            wh[kk] = fire_write(kk)
        wh.pop(_NCHUNK - 1).wait()

    return k(coords, pfpad)


def kernel(pillar_features, coords):
    pfpad = jnp.concatenate(
        [pillar_features, jnp.zeros((_PPAD, _C), jnp.float32)], axis=0)
    cflat = coords.astype(jnp.int32).reshape(-1)
    out_t, _ = _sc_scatter(cflat, pfpad)
    return out_t.T.reshape(1, _C, _NY, _NX)
